# Initial kernel scaffold; baseline (speedup 1.0000x reference)
import functools

import jax
import jax.numpy as jnp
from jax import lax
from jax.experimental import pallas as pl
from jax.experimental.pallas import tpu as pltpu
from jax.experimental.pallas import tpu_sc as plsc

N = 10000
F = 128
NPAD = 10240
CHUNK = 128
NCH = 80
NT = 32
EPAD = NT * NCH * CHUNK
SHARE = NPAD // 16

_MESH = plsc.VectorSubcoreMesh(core_axis_name="c", subcore_axis_name="s")


@functools.partial(
    pl.kernel,
    out_type=jax.ShapeDtypeStruct((2, NPAD, 16), jnp.float32),
    mesh=_MESH,
    scratch_types=[
        pltpu.VMEM((NCH, CHUNK), jnp.int32),
        pltpu.VMEM((CHUNK, 16), jnp.float32),
        pltpu.VMEM((SHARE, 16), jnp.float32),
        pltpu.VMEM_SHARED((NPAD, 16), jnp.float32),
    ],
)
def _sc_degree(dst_hbm, ones_hbm, zeros_hbm, out_hbm, dst_v, ones_v, z_v, acc_sh):
    cid = lax.axis_index("c")
    sid = lax.axis_index("s")
    wid = cid * 16 + sid
    pltpu.sync_copy(dst_hbm.at[wid], dst_v)
    pltpu.sync_copy(ones_hbm, ones_v)
    pltpu.sync_copy(zeros_hbm, z_v)
    pltpu.sync_copy(z_v, acc_sh.at[pl.ds(sid * SHARE, SHARE)])
    plsc.subcore_barrier()

    def body(j, carry):
        pltpu.sync_copy(ones_v, acc_sh.at[dst_v.at[j]], add=True)
        return carry

    lax.fori_loop(0, NCH, body, 0)
    plsc.subcore_barrier()
    pltpu.sync_copy(acc_sh.at[pl.ds(sid * SHARE, SHARE)],
                    out_hbm.at[cid, pl.ds(sid * SHARE, SHARE)])


@functools.partial(
    pl.kernel,
    out_type=jax.ShapeDtypeStruct((2, NPAD, F), jnp.float32),
    mesh=_MESH,
    scratch_types=[
        pltpu.VMEM((NCH, CHUNK), jnp.int32),
        pltpu.VMEM((NCH, CHUNK), jnp.int32),
        pltpu.VMEM((CHUNK, F), jnp.float32),
        pltpu.VMEM((CHUNK, F), jnp.float32),
        pltpu.VMEM_SHARED((NPAD, F), jnp.float32),
    ],
)
def _sc_propagate(y_hbm, src_hbm, dst_hbm, zeros_hbm, out_hbm,
                  src_v, dst_v, rows_v, z_v, acc_sh):
    cid = lax.axis_index("c")
    sid = lax.axis_index("s")
    wid = cid * 16 + sid
    pltpu.sync_copy(src_hbm.at[wid], src_v)
    pltpu.sync_copy(dst_hbm.at[wid], dst_v)
    pltpu.sync_copy(zeros_hbm, z_v)
    for k in range(SHARE // CHUNK):
        pltpu.sync_copy(z_v, acc_sh.at[pl.ds(sid * SHARE + k * CHUNK, CHUNK)])
    plsc.subcore_barrier()

    def body(j, carry):
        pltpu.sync_copy(y_hbm.at[src_v.at[j]], rows_v)
        pltpu.sync_copy(rows_v, acc_sh.at[dst_v.at[j]], add=True)
        return carry

    lax.fori_loop(0, NCH, body, 0)
    plsc.subcore_barrier()
    pltpu.sync_copy(acc_sh.at[pl.ds(sid * SHARE, SHARE)],
                    out_hbm.at[cid, pl.ds(sid * SHARE, SHARE)])


BN = 640
BNO = 400


def _norm_from(deg_ref):
    d = deg_ref[0, :, 0:1] + deg_ref[1, :, 0:1]
    return lax.rsqrt(jnp.maximum(d, 1.0))


def _tc_y1_body(deg_ref, feat_ref, y_ref):
    y_ref[...] = feat_ref[...] * _norm_from(deg_ref)


_tc_y1 = pl.pallas_call(
    _tc_y1_body,
    grid=(NPAD // BN,),
    in_specs=[
        pl.BlockSpec((2, BN, 16), lambda i: (0, i, 0)),
        pl.BlockSpec((BN, F), lambda i: (i, 0)),
    ],
    out_specs=pl.BlockSpec((BN, F), lambda i: (i, 0)),
    out_shape=jax.ShapeDtypeStruct((NPAD, F), jnp.float32),
)


def _tc_tx1_body(r_ref, deg_ref, feat_ref, hp_ref, tx1_ref, y2_ref):
    r = r_ref[0, 0]
    nrm = _norm_from(deg_ref)
    h = (hp_ref[0] + hp_ref[1]) * nrm
    tx1 = (r - 1.0) * feat_ref[...] - r * h
    tx1_ref[...] = tx1
    y2_ref[...] = tx1 * nrm


_tc_tx1 = pl.pallas_call(
    _tc_tx1_body,
    grid=(NPAD // BN,),
    in_specs=[
        pl.BlockSpec(memory_space=pltpu.SMEM),
        pl.BlockSpec((2, BN, 16), lambda i: (0, i, 0)),
        pl.BlockSpec((BN, F), lambda i: (i, 0)),
        pl.BlockSpec((2, BN, F), lambda i: (0, i, 0)),
    ],
    out_specs=[
        pl.BlockSpec((BN, F), lambda i: (i, 0)),
        pl.BlockSpec((BN, F), lambda i: (i, 0)),
    ],
    out_shape=[
        jax.ShapeDtypeStruct((NPAD, F), jnp.float32),
        jax.ShapeDtypeStruct((NPAD, F), jnp.float32),
    ],
)


def _tc_out_body(r_ref, deg_ref, feat_ref, tx1_ref, hp_ref, w_ref, out_ref):
    r = r_ref[0, 0]
    nrm = _norm_from(deg_ref)
    h2 = (hp_ref[0] + hp_ref[1]) * nrm
    f = feat_ref[...]
    t1 = tx1_ref[...]
    t2 = -2.0 * r * h2 + 2.0 * (r - 1.0) * t1 - f
    x = jnp.concatenate([f, t1, t2], axis=1)
    out_ref[...] = jnp.dot(x, w_ref[...], preferred_element_type=jnp.float32)


_tc_out = pl.pallas_call(
    _tc_out_body,
    grid=(N // BNO,),
    in_specs=[
        pl.BlockSpec(memory_space=pltpu.SMEM),
        pl.BlockSpec((2, BNO, 16), lambda i: (0, i, 0)),
        pl.BlockSpec((BNO, F), lambda i: (i, 0)),
        pl.BlockSpec((BNO, F), lambda i: (i, 0)),
        pl.BlockSpec((2, BNO, F), lambda i: (0, i, 0)),
        pl.BlockSpec((3 * F, F), lambda i: (0, 0)),
    ],
    out_specs=pl.BlockSpec((BNO, F), lambda i: (i, 0)),
    out_shape=jax.ShapeDtypeStruct((N, F), jnp.float32),
)


def kernel(feat, edge_index, lambda_max, W0, W1, W2):
    src = edge_index[0].astype(jnp.int32)
    dst = edge_index[1].astype(jnp.int32)
    e = src.shape[0]
    pad = jnp.full((EPAD - e,), N, jnp.int32)
    src_t = jnp.concatenate([src, pad]).reshape(NT, NCH, CHUNK)
    dst_t = jnp.concatenate([dst, pad]).reshape(NT, NCH, CHUNK)
    feat_pad = jnp.pad(feat, ((0, NPAD - N), (0, 0)))
    ones16 = jnp.ones((CHUNK, 16), jnp.float32)
    zeros16 = jnp.zeros((SHARE, 16), jnp.float32)
    zeros128 = jnp.zeros((CHUNK, F), jnp.float32)
    r = jnp.reshape((2.0 / lambda_max).astype(jnp.float32), (1, 1))

    deg2 = _sc_degree(dst_t, ones16, zeros16)
    y1 = _tc_y1(deg2, feat_pad)
    h1 = _sc_propagate(y1, src_t, dst_t, zeros128)
    tx1, y2 = _tc_tx1(r, deg2, feat_pad, h1)
    h2 = _sc_propagate(y2, src_t, dst_t, zeros128)
    wcat = jnp.concatenate([W0, W1, W2], axis=0)
    return _tc_out(r, deg2, feat, tx1, h2, wcat)


# speedup vs baseline: 3.1307x; 3.1307x over previous
import functools

import jax
import jax.numpy as jnp
from jax import lax
from jax.experimental import pallas as pl
from jax.experimental.pallas import tpu as pltpu
from jax.experimental.pallas import tpu_sc as plsc

N = 10000
F = 128
NPAD = 10240
CHUNK = 128
NCH = 80
NT = 32
EPAD = NT * NCH * CHUNK
SHARE = NPAD // 16

_MESH = plsc.VectorSubcoreMesh(core_axis_name="c", subcore_axis_name="s")


@functools.partial(
    pl.kernel,
    out_type=jax.ShapeDtypeStruct((2, NPAD, F), jnp.float32),
    mesh=_MESH,
    scratch_types=[
        pltpu.VMEM((NCH, CHUNK), jnp.int32),
        pltpu.VMEM((CHUNK, F), jnp.float32),
        pltpu.VMEM_SHARED((NPAD, F), jnp.float32),
    ],
)
def _sc_degree(dst_hbm, ones_hbm, zeros_hbm, out_hbm, dst_v, buf_v, acc_sh):
    cid = lax.axis_index("c")
    sid = lax.axis_index("s")
    wid = cid * 16 + sid
    pltpu.sync_copy(dst_hbm.at[wid], dst_v)
    pltpu.sync_copy(zeros_hbm, buf_v)
    for k in range(SHARE // CHUNK):
        pltpu.sync_copy(buf_v, acc_sh.at[pl.ds(sid * SHARE + k * CHUNK, CHUNK)])
    plsc.subcore_barrier()
    pltpu.sync_copy(ones_hbm, buf_v)

    def body(j, carry):
        pltpu.sync_copy(buf_v, acc_sh.at[dst_v.at[j]], add=True)
        return carry

    lax.fori_loop(0, NCH, body, 0)
    plsc.subcore_barrier()
    for k in range(SHARE // CHUNK):
        sl = pl.ds(sid * SHARE + k * CHUNK, CHUNK)
        pltpu.sync_copy(acc_sh.at[sl], buf_v)
        pltpu.sync_copy(buf_v, out_hbm.at[cid, sl])


@functools.partial(
    pl.kernel,
    out_type=jax.ShapeDtypeStruct((2, NPAD, F), jnp.float32),
    mesh=_MESH,
    scratch_types=[
        pltpu.VMEM((NCH, CHUNK), jnp.int32),
        pltpu.VMEM((NCH, CHUNK), jnp.int32),
        pltpu.VMEM((CHUNK, F), jnp.float32),
        pltpu.VMEM_SHARED((NPAD, F), jnp.float32),
    ],
)
def _sc_propagate(y_hbm, src_hbm, dst_hbm, zeros_hbm, out_hbm,
                  src_v, dst_v, rows_v, acc_sh):
    cid = lax.axis_index("c")
    sid = lax.axis_index("s")
    wid = cid * 16 + sid
    pltpu.sync_copy(src_hbm.at[wid], src_v)
    pltpu.sync_copy(dst_hbm.at[wid], dst_v)
    pltpu.sync_copy(zeros_hbm, rows_v)
    for k in range(SHARE // CHUNK):
        pltpu.sync_copy(rows_v, acc_sh.at[pl.ds(sid * SHARE + k * CHUNK, CHUNK)])
    plsc.subcore_barrier()

    def body(j, carry):
        pltpu.sync_copy(y_hbm.at[src_v.at[j]], rows_v)
        pltpu.sync_copy(rows_v, acc_sh.at[dst_v.at[j]], add=True)
        return carry

    lax.fori_loop(0, NCH, body, 0)
    plsc.subcore_barrier()
    for k in range(SHARE // CHUNK):
        sl = pl.ds(sid * SHARE + k * CHUNK, CHUNK)
        pltpu.sync_copy(acc_sh.at[sl], rows_v)
        pltpu.sync_copy(rows_v, out_hbm.at[cid, sl])


BN = 640
BNO = 400


def _norm_from(deg_ref):
    d = deg_ref[0, :, 0:1] + deg_ref[1, :, 0:1]
    return lax.rsqrt(jnp.maximum(d, 1.0))


def _tc_y1_body(deg_ref, feat_ref, y_ref):
    y_ref[...] = feat_ref[...] * _norm_from(deg_ref)


_tc_y1 = pl.pallas_call(
    _tc_y1_body,
    grid=(NPAD // BN,),
    in_specs=[
        pl.BlockSpec((2, BN, F), lambda i: (0, i, 0)),
        pl.BlockSpec((BN, F), lambda i: (i, 0)),
    ],
    out_specs=pl.BlockSpec((BN, F), lambda i: (i, 0)),
    out_shape=jax.ShapeDtypeStruct((NPAD, F), jnp.float32),
)


def _tc_tx1_body(r_ref, deg_ref, feat_ref, hp_ref, tx1_ref, y2_ref):
    r = r_ref[0, 0]
    nrm = _norm_from(deg_ref)
    h = (hp_ref[0] + hp_ref[1]) * nrm
    tx1 = (r - 1.0) * feat_ref[...] - r * h
    tx1_ref[...] = tx1
    y2_ref[...] = tx1 * nrm


_tc_tx1 = pl.pallas_call(
    _tc_tx1_body,
    grid=(NPAD // BN,),
    in_specs=[
        pl.BlockSpec(memory_space=pltpu.SMEM),
        pl.BlockSpec((2, BN, F), lambda i: (0, i, 0)),
        pl.BlockSpec((BN, F), lambda i: (i, 0)),
        pl.BlockSpec((2, BN, F), lambda i: (0, i, 0)),
    ],
    out_specs=[
        pl.BlockSpec((BN, F), lambda i: (i, 0)),
        pl.BlockSpec((BN, F), lambda i: (i, 0)),
    ],
    out_shape=[
        jax.ShapeDtypeStruct((NPAD, F), jnp.float32),
        jax.ShapeDtypeStruct((NPAD, F), jnp.float32),
    ],
)


def _tc_out_body(r_ref, deg_ref, feat_ref, tx1_ref, hp_ref, w_ref, out_ref):
    r = r_ref[0, 0]
    nrm = _norm_from(deg_ref)
    h2 = (hp_ref[0] + hp_ref[1]) * nrm
    f = feat_ref[...]
    t1 = tx1_ref[...]
    t2 = -2.0 * r * h2 + 2.0 * (r - 1.0) * t1 - f
    x = jnp.concatenate([f, t1, t2], axis=1)
    out_ref[...] = jnp.dot(x, w_ref[...], preferred_element_type=jnp.float32)


_tc_out = pl.pallas_call(
    _tc_out_body,
    grid=(N // BNO,),
    in_specs=[
        pl.BlockSpec(memory_space=pltpu.SMEM),
        pl.BlockSpec((2, BNO, F), lambda i: (0, i, 0)),
        pl.BlockSpec((BNO, F), lambda i: (i, 0)),
        pl.BlockSpec((BNO, F), lambda i: (i, 0)),
        pl.BlockSpec((2, BNO, F), lambda i: (0, i, 0)),
        pl.BlockSpec((3 * F, F), lambda i: (0, 0)),
    ],
    out_specs=pl.BlockSpec((BNO, F), lambda i: (i, 0)),
    out_shape=jax.ShapeDtypeStruct((N, F), jnp.float32),
)


def kernel(feat, edge_index, lambda_max, W0, W1, W2):
    src = edge_index[0].astype(jnp.int32)
    dst = edge_index[1].astype(jnp.int32)
    e = src.shape[0]
    pad = jnp.full((EPAD - e,), N, jnp.int32)
    src_t = jnp.concatenate([src, pad]).reshape(NT, NCH, CHUNK)
    dst_t = jnp.concatenate([dst, pad]).reshape(NT, NCH, CHUNK)
    feat_pad = jnp.pad(feat, ((0, NPAD - N), (0, 0)))
    ones128 = jnp.ones((CHUNK, F), jnp.float32)
    zeros128 = jnp.zeros((CHUNK, F), jnp.float32)
    r = jnp.reshape((2.0 / lambda_max).astype(jnp.float32), (1, 1))

    deg2 = _sc_degree(dst_t, ones128, zeros128)
    y1 = _tc_y1(deg2, feat_pad)
    h1 = _sc_propagate(y1, src_t, dst_t, zeros128)
    tx1, y2 = _tc_tx1(r, deg2, feat_pad, h1)
    h2 = _sc_propagate(y2, src_t, dst_t, zeros128)
    wcat = jnp.concatenate([W0, W1, W2], axis=0)
    return _tc_out(r, deg2, feat, tx1, h2, wcat)


# speedup vs baseline: 3.5089x; 1.1208x over previous
import functools

import jax
import jax.numpy as jnp
from jax import lax
from jax.experimental import pallas as pl
from jax.experimental.pallas import tpu as pltpu
from jax.experimental.pallas import tpu_sc as plsc

N = 10000
F = 128
NPAD = 10240
CHUNK = 128
NCH = 80
NT = 32
EPAD = NT * NCH * CHUNK
SHARE = NPAD // 16

_MESH = plsc.VectorSubcoreMesh(core_axis_name="c", subcore_axis_name="s")


@functools.partial(
    pl.kernel,
    out_type=jax.ShapeDtypeStruct((2, NPAD, F), jnp.float32),
    mesh=_MESH,
    scratch_types=[
        pltpu.VMEM((NCH, CHUNK), jnp.int32),
        pltpu.VMEM((CHUNK, F), jnp.float32),
        pltpu.VMEM_SHARED((NPAD, F), jnp.float32),
    ],
)
def _sc_degree(dst_hbm, ones_hbm, zeros_hbm, out_hbm, dst_v, buf_v, acc_sh):
    cid = lax.axis_index("c")
    sid = lax.axis_index("s")
    wid = cid * 16 + sid
    pltpu.sync_copy(dst_hbm.at[wid], dst_v)
    pltpu.sync_copy(zeros_hbm, buf_v)
    for k in range(SHARE // CHUNK):
        pltpu.sync_copy(buf_v, acc_sh.at[pl.ds(sid * SHARE + k * CHUNK, CHUNK)])
    plsc.subcore_barrier()
    pltpu.sync_copy(ones_hbm, buf_v)

    def body(j, carry):
        pltpu.sync_copy(buf_v, acc_sh.at[dst_v.at[j]], add=True)
        return carry

    lax.fori_loop(0, NCH, body, 0)
    plsc.subcore_barrier()
    for k in range(SHARE // CHUNK):
        sl = pl.ds(sid * SHARE + k * CHUNK, CHUNK)
        pltpu.sync_copy(acc_sh.at[sl], buf_v)
        pltpu.sync_copy(buf_v, out_hbm.at[cid, sl])


@functools.partial(
    pl.kernel,
    out_type=jax.ShapeDtypeStruct((2, NPAD, F), jnp.float32),
    mesh=_MESH,
    scratch_types=[
        pltpu.VMEM((NCH, CHUNK), jnp.int32),
        pltpu.VMEM((2, CHUNK), jnp.int32),
        pltpu.VMEM((CHUNK, F), jnp.float32),
        pltpu.VMEM((CHUNK, F), jnp.float32),
        pltpu.SemaphoreType.DMA,
        pltpu.SemaphoreType.DMA,
        pltpu.SemaphoreType.DMA,
        pltpu.SemaphoreType.DMA,
        pltpu.VMEM_SHARED((NPAD, F), jnp.float32),
    ],
)
def _sc_propagate(y_hbm, src_hbm, dst_hbm, zeros_hbm, out_hbm,
                  src_v, dring, rows0, rows1, sg0, sg1, sd0, sd1, acc_sh):
    cid = lax.axis_index("c")
    sid = lax.axis_index("s")
    wid = cid * 16 + sid
    pltpu.sync_copy(src_hbm.at[wid], src_v)
    pltpu.sync_copy(zeros_hbm, rows0)
    for k in range(SHARE // CHUNK):
        pltpu.sync_copy(rows0, acc_sh.at[pl.ds(sid * SHARE + k * CHUNK, CHUNK)])
    plsc.subcore_barrier()

    pltpu.async_copy(dst_hbm.at[wid, 0], dring.at[0], sd0)
    pltpu.async_copy(dst_hbm.at[wid, 1], dring.at[1], sd1)
    pltpu.async_copy(y_hbm.at[src_v.at[0]], rows0, sg0)
    pltpu.async_copy(y_hbm.at[src_v.at[1]], rows1, sg1)

    def body(p, carry):
        j0 = 2 * p
        j1 = 2 * p + 1
        n0 = jnp.minimum(j0 + 2, NCH - 1)
        n1 = jnp.minimum(j1 + 2, NCH - 1)
        pltpu.make_async_copy(y_hbm.at[src_v.at[j0]], rows0, sg0).wait()
        pltpu.make_async_copy(dst_hbm.at[wid, j0], dring.at[0], sd0).wait()
        pltpu.sync_copy(rows0, acc_sh.at[dring.at[0]], add=True)
        pltpu.async_copy(dst_hbm.at[wid, n0], dring.at[0], sd0)
        pltpu.async_copy(y_hbm.at[src_v.at[n0]], rows0, sg0)
        pltpu.make_async_copy(y_hbm.at[src_v.at[j1]], rows1, sg1).wait()
        pltpu.make_async_copy(dst_hbm.at[wid, j1], dring.at[1], sd1).wait()
        pltpu.sync_copy(rows1, acc_sh.at[dring.at[1]], add=True)
        pltpu.async_copy(dst_hbm.at[wid, n1], dring.at[1], sd1)
        pltpu.async_copy(y_hbm.at[src_v.at[n1]], rows1, sg1)
        return carry

    lax.fori_loop(0, NCH // 2, body, 0)
    pltpu.make_async_copy(y_hbm.at[src_v.at[NCH - 1]], rows0, sg0).wait()
    pltpu.make_async_copy(y_hbm.at[src_v.at[NCH - 1]], rows1, sg1).wait()
    pltpu.make_async_copy(dst_hbm.at[wid, NCH - 1], dring.at[0], sd0).wait()
    pltpu.make_async_copy(dst_hbm.at[wid, NCH - 1], dring.at[1], sd1).wait()
    plsc.subcore_barrier()
    for k in range(SHARE // CHUNK):
        sl = pl.ds(sid * SHARE + k * CHUNK, CHUNK)
        pltpu.sync_copy(acc_sh.at[sl], rows0)
        pltpu.sync_copy(rows0, out_hbm.at[cid, sl])


BN = 640
BNO = 400


def _norm_from(deg_ref):
    d = deg_ref[0, :, 0:1] + deg_ref[1, :, 0:1]
    return lax.rsqrt(jnp.maximum(d, 1.0))


def _tc_y1_body(deg_ref, feat_ref, y_ref):
    y_ref[...] = feat_ref[...] * _norm_from(deg_ref)


_tc_y1 = pl.pallas_call(
    _tc_y1_body,
    grid=(NPAD // BN,),
    in_specs=[
        pl.BlockSpec((2, BN, F), lambda i: (0, i, 0)),
        pl.BlockSpec((BN, F), lambda i: (i, 0)),
    ],
    out_specs=pl.BlockSpec((BN, F), lambda i: (i, 0)),
    out_shape=jax.ShapeDtypeStruct((NPAD, F), jnp.float32),
)


def _tc_tx1_body(r_ref, deg_ref, feat_ref, hp_ref, tx1_ref, y2_ref):
    r = r_ref[0, 0]
    nrm = _norm_from(deg_ref)
    h = (hp_ref[0] + hp_ref[1]) * nrm
    tx1 = (r - 1.0) * feat_ref[...] - r * h
    tx1_ref[...] = tx1
    y2_ref[...] = tx1 * nrm


_tc_tx1 = pl.pallas_call(
    _tc_tx1_body,
    grid=(NPAD // BN,),
    in_specs=[
        pl.BlockSpec(memory_space=pltpu.SMEM),
        pl.BlockSpec((2, BN, F), lambda i: (0, i, 0)),
        pl.BlockSpec((BN, F), lambda i: (i, 0)),
        pl.BlockSpec((2, BN, F), lambda i: (0, i, 0)),
    ],
    out_specs=[
        pl.BlockSpec((BN, F), lambda i: (i, 0)),
        pl.BlockSpec((BN, F), lambda i: (i, 0)),
    ],
    out_shape=[
        jax.ShapeDtypeStruct((NPAD, F), jnp.float32),
        jax.ShapeDtypeStruct((NPAD, F), jnp.float32),
    ],
)


def _tc_out_body(r_ref, deg_ref, feat_ref, tx1_ref, hp_ref, w_ref, out_ref):
    r = r_ref[0, 0]
    nrm = _norm_from(deg_ref)
    h2 = (hp_ref[0] + hp_ref[1]) * nrm
    f = feat_ref[...]
    t1 = tx1_ref[...]
    t2 = -2.0 * r * h2 + 2.0 * (r - 1.0) * t1 - f
    x = jnp.concatenate([f, t1, t2], axis=1)
    out_ref[...] = jnp.dot(x, w_ref[...], preferred_element_type=jnp.float32)


_tc_out = pl.pallas_call(
    _tc_out_body,
    grid=(N // BNO,),
    in_specs=[
        pl.BlockSpec(memory_space=pltpu.SMEM),
        pl.BlockSpec((2, BNO, F), lambda i: (0, i, 0)),
        pl.BlockSpec((BNO, F), lambda i: (i, 0)),
        pl.BlockSpec((BNO, F), lambda i: (i, 0)),
        pl.BlockSpec((2, BNO, F), lambda i: (0, i, 0)),
        pl.BlockSpec((3 * F, F), lambda i: (0, 0)),
    ],
    out_specs=pl.BlockSpec((BNO, F), lambda i: (i, 0)),
    out_shape=jax.ShapeDtypeStruct((N, F), jnp.float32),
)


def kernel(feat, edge_index, lambda_max, W0, W1, W2):
    src = edge_index[0].astype(jnp.int32)
    dst = edge_index[1].astype(jnp.int32)
    e = src.shape[0]
    pad = jnp.full((EPAD - e,), N, jnp.int32)
    src_t = jnp.concatenate([src, pad]).reshape(NT, NCH, CHUNK)
    dst_t = jnp.concatenate([dst, pad]).reshape(NT, NCH, CHUNK)
    feat_pad = jnp.pad(feat, ((0, NPAD - N), (0, 0)))
    ones128 = jnp.ones((CHUNK, F), jnp.float32)
    zeros128 = jnp.zeros((CHUNK, F), jnp.float32)
    r = jnp.reshape((2.0 / lambda_max).astype(jnp.float32), (1, 1))

    deg2 = _sc_degree(dst_t, ones128, zeros128)
    y1 = _tc_y1(deg2, feat_pad)
    h1 = _sc_propagate(y1, src_t, dst_t, zeros128)
    tx1, y2 = _tc_tx1(r, deg2, feat_pad, h1)
    h2 = _sc_propagate(y2, src_t, dst_t, zeros128)
    wcat = jnp.concatenate([W0, W1, W2], axis=0)
    return _tc_out(r, deg2, feat, tx1, h2, wcat)
